# baseline (speedup 1.0000x reference)
import functools

import jax
import jax.numpy as jnp
from jax import lax
from jax.experimental import pallas as pl
from jax.experimental.pallas import tpu as pltpu
from jax.experimental.pallas import tpu_sc as plsc

N = 10000
D = 128
N_PAD = 10240
BLK = 512

NC = 2
NS = 16
CH = 128
ROWS_PER_TILE = N_PAD // NS


def _leaky(x):
    return jnp.where(x > 0, x, 0.1 * x)


def _dot3(a, b):
    return jnp.dot(a.astype(jnp.bfloat16), b.astype(jnp.bfloat16),
                   preferred_element_type=jnp.float32)


def _mlp_kernel(x_ref, w1_ref, b1_ref, w2_ref, b2_ref, o_ref):
    h = _leaky(_dot3(x_ref[...], w1_ref[...]) + b1_ref[...])
    o = _dot3(h, w2_ref[...]) + b2_ref[...]
    o_ref[...] = o.astype(jnp.bfloat16)


def _mlp(x, w1, b1, w2, b2):
    rows, din = x.shape
    dh = w1.shape[1]
    return pl.pallas_call(
        _mlp_kernel,
        grid=(rows // BLK,),
        in_specs=[
            pl.BlockSpec((BLK, din), lambda i: (i, 0)),
            pl.BlockSpec((din, dh), lambda i: (0, 0)),
            pl.BlockSpec((1, dh), lambda i: (0, 0)),
            pl.BlockSpec((dh, D), lambda i: (0, 0)),
            pl.BlockSpec((1, D), lambda i: (0, 0)),
        ],
        out_specs=pl.BlockSpec((BLK, D), lambda i: (i, 0)),
        out_shape=jax.ShapeDtypeStruct((rows, D), jnp.bfloat16),
    )(x, w1, b1.reshape(1, -1), w2, b2.reshape(1, -1))


EBLK = 1280


def _edge_mlp(ea, w1, b1, w2, b2, scale):
    w2s = w2 * scale[0]
    b2s = b2 * scale[0]
    rows, din = ea.shape
    dh = w1.shape[1]
    return pl.pallas_call(
        _mlp_kernel,
        grid=(rows // EBLK,),
        in_specs=[
            pl.BlockSpec((EBLK, din), lambda i: (i, 0)),
            pl.BlockSpec((din, dh), lambda i: (0, 0)),
            pl.BlockSpec((1, dh), lambda i: (0, 0)),
            pl.BlockSpec((dh, D), lambda i: (0, 0)),
            pl.BlockSpec((1, D), lambda i: (0, 0)),
        ],
        out_specs=pl.BlockSpec((EBLK, D), lambda i: (i, 0)),
        out_shape=jax.ShapeDtypeStruct((rows, D), jnp.bfloat16),
    )(ea, w1, b1.reshape(1, -1), w2s, b2s.reshape(1, -1))


def _ln_kernel(x_ref, a0_ref, a1_ref, rw_ref, g_ref, b_ref, o_ref):
    agg = a0_ref[...].astype(jnp.float32) + a1_ref[...].astype(jnp.float32)
    out = x_ref[...] + agg * rw_ref[0, 0]
    mean = jnp.mean(out, axis=1, keepdims=True)
    cen = out - mean
    var = jnp.mean(cen * cen, axis=1, keepdims=True)
    o_ref[...] = cen * jax.lax.rsqrt(var + 1e-5) * g_ref[...] + b_ref[...]


def _ln_residual(x, a0, a1, rw, gamma, beta):
    rows = x.shape[0]
    return pl.pallas_call(
        _ln_kernel,
        grid=(rows // BLK,),
        in_specs=[
            pl.BlockSpec((BLK, D), lambda i: (i, 0)),
            pl.BlockSpec((BLK, D), lambda i: (i, 0)),
            pl.BlockSpec((BLK, D), lambda i: (i, 0)),
            pl.BlockSpec((1, 1), lambda i: (0, 0)),
            pl.BlockSpec((1, D), lambda i: (0, 0)),
            pl.BlockSpec((1, D), lambda i: (0, 0)),
        ],
        out_specs=pl.BlockSpec((BLK, D), lambda i: (i, 0)),
        out_shape=jax.ShapeDtypeStruct((rows, D), jnp.float32),
    )(x, a0, a1, rw.reshape(1, 1), gamma.reshape(1, -1), beta.reshape(1, -1))


def _sc_agg(xt, te, sd):
    nchunk = sd.shape[0]
    ncc = nchunk // NC
    pairs = ((ncc + NS - 1) // NS + 1) // 2
    mesh = plsc.VectorSubcoreMesh(core_axis_name="c", subcore_axis_name="s")

    @functools.partial(
        pl.kernel,
        out_type=jax.ShapeDtypeStruct((NC, N_PAD, D), jnp.bfloat16),
        mesh=mesh,
        scratch_types=[
            pltpu.VMEM((2, 2, CH), jnp.int32),
            pltpu.VMEM((2, CH, D), jnp.bfloat16),
            pltpu.VMEM((2, CH, D), jnp.bfloat16),
            pltpu.VMEM((CH, D), jnp.bfloat16),
            pltpu.VMEM_SHARED((N_PAD, D), jnp.bfloat16),
            pltpu.SemaphoreType.DMA,
            pltpu.SemaphoreType.DMA,
            pltpu.SemaphoreType.DMA,
            pltpu.SemaphoreType.DMA,
            pltpu.SemaphoreType.DMA,
            pltpu.SemaphoreType.DMA,
        ],
        compiler_params=pltpu.CompilerParams(use_tc_tiling_on_sc=False),
    )
    def k(xt_hbm, te_hbm, sd_hbm, out_hbm, sdb, rows, tev, zbuf, table,
          si0, si1, sg0, sg1, st0, st1):
        c = lax.axis_index("c")
        s = lax.axis_index("s")
        base = c * ncc
        sem_i = [si0, si1]
        sem_g = [sg0, sg1]
        sem_t = [st0, st1]

        @pl.loop(0, CH)
        def _(r):
            @pl.loop(0, D // 32)
            def _(j):
                zbuf[r, pl.ds(j * 32, 32)] = jnp.zeros((32,), jnp.bfloat16)

        @pl.loop(0, ROWS_PER_TILE // CH)
        def _(kk):
            pltpu.sync_copy(zbuf, table.at[pl.ds(s * ROWS_PER_TILE + kk * CH, CH)])

        plsc.subcore_barrier()

        def load_idx(b, g):
            pltpu.async_copy(sd_hbm.at[base + g], sdb.at[b], sem_i[b])

        def wait_idx(b):
            pltpu.make_async_copy(sd_hbm.at[0], sdb.at[b], sem_i[b]).wait()

        def load_data(b, g):
            pltpu.async_copy(xt_hbm.at[sdb.at[b, 0]], rows.at[b], sem_g[b])
            pltpu.async_copy(te_hbm.at[pl.ds((base + g) * CH, CH)], tev.at[b],
                             sem_t[b])

        def wait_data(b):
            pltpu.make_async_copy(xt_hbm.at[pl.ds(0, CH)], rows.at[b],
                                  sem_g[b]).wait()
            pltpu.make_async_copy(te_hbm.at[pl.ds(0, CH)], tev.at[b],
                                  sem_t[b]).wait()

        def compute(b):
            @pl.loop(0, CH)
            def _(r):
                for j in range(D // 32):
                    sl = pl.ds(j * 32, 32)
                    rows[b, r, sl] = rows[b, r, sl] * tev[b, r, sl]

            pltpu.sync_copy(rows.at[b], table.at[sdb.at[b, 1]], add=True)

        load_idx(0, s)

        @pl.when(NS + s < ncc)
        def _():
            load_idx(1, NS + s)

        wait_idx(0)
        load_data(0, s)

        @pl.loop(0, pairs)
        def _(ip):
            for b in range(2):
                g = (2 * ip + b) * NS + s
                g1 = g + NS
                g2 = g1 + NS

                @pl.when(g < ncc)
                def _():
                    @pl.when(g1 < ncc)
                    def _():
                        wait_idx(1 - b)
                        load_data(1 - b, g1)

                    wait_data(b)
                    compute(b)

                    @pl.when(g2 < ncc)
                    def _():
                        load_idx(b, g2)

        plsc.subcore_barrier()

        @pl.loop(0, ROWS_PER_TILE // CH)
        def _(kk):
            r0 = s * ROWS_PER_TILE + kk * CH
            pltpu.sync_copy(table.at[pl.ds(r0, CH)], out_hbm.at[c, pl.ds(r0, CH)])

    return k(xt, te, sd)


def kernel(x, edge_index, edge_attr, nt_W1, nt_b1, nt_W2, nt_b2,
           et_W1, et_b1, et_W2, et_b2, residual_weight, dot_product_scale,
           ln_gamma, ln_beta):
    e = edge_index.shape[1]
    sd = edge_index.astype(jnp.int32).reshape(2, e // CH, CH).transpose(1, 0, 2)
    x_p = jnp.pad(x, ((0, N_PAD - N), (0, 0)))

    xt = _mlp(x_p, nt_W1, nt_b1, nt_W2, nt_b2)
    te = _edge_mlp(edge_attr, et_W1, et_b1, et_W2, et_b2, dot_product_scale)

    agg = _sc_agg(xt, te, sd)

    out = _ln_residual(x_p, agg[0], agg[1], residual_weight, ln_gamma, ln_beta)
    return out[:N]


# speedup vs baseline: 1.2894x; 1.2894x over previous
import functools

import jax
import jax.numpy as jnp
from jax import lax
from jax.experimental import pallas as pl
from jax.experimental.pallas import tpu as pltpu
from jax.experimental.pallas import tpu_sc as plsc

N = 10000
D = 128
N_PAD = 10240
BLK = 512

NC = 2
NS = 16
CH = 128
DH = D // NC
ROWS_PER_TILE = N_PAD // NS


def _leaky(x):
    return jnp.where(x > 0, x, 0.1 * x)


def _dot3(a, b):
    return jnp.dot(a.astype(jnp.bfloat16), b.astype(jnp.bfloat16),
                   preferred_element_type=jnp.float32)


def _mlp_kernel(x_ref, w1_ref, b1_ref, w2_ref, b2_ref, s_ref, o0_ref, o1_ref):
    h = _leaky(_dot3(x_ref[...], w1_ref[...]) + b1_ref[...])
    o = _dot3(h, w2_ref[...]) + b2_ref[...]
    o = o * s_ref[0, 0]
    o0_ref[...] = o[:, :DH]
    o1_ref[...] = o[:, DH:]


def _mlp(x, w1, b1, w2, b2, scale):
    rows, din = x.shape
    dh = w1.shape[1]
    grid = rows // BLK
    return pl.pallas_call(
        _mlp_kernel,
        grid=(grid,),
        in_specs=[
            pl.BlockSpec((BLK, din), lambda i: (i, 0)),
            pl.BlockSpec((din, dh), lambda i: (0, 0)),
            pl.BlockSpec((1, dh), lambda i: (0, 0)),
            pl.BlockSpec((dh, D), lambda i: (0, 0)),
            pl.BlockSpec((1, D), lambda i: (0, 0)),
            pl.BlockSpec((1, 1), lambda i: (0, 0)),
        ],
        out_specs=[pl.BlockSpec((BLK, DH), lambda i: (i, 0))] * 2,
        out_shape=[jax.ShapeDtypeStruct((rows, DH), jnp.float32)] * 2,
    )(x, w1, b1.reshape(1, -1), w2, b2.reshape(1, -1), scale.reshape(1, 1))


EBLK = 1280


def _edge_mlp_kernel(x_ref, w1_ref, b1_ref, w20_ref, w21_ref, b20_ref,
                     b21_ref, o0_ref, o1_ref):
    h = _leaky(_dot3(x_ref[...], w1_ref[...]) + b1_ref[...])
    o0_ref[...] = _dot3(h, w20_ref[...]) + b20_ref[...]
    o1_ref[...] = _dot3(h, w21_ref[...]) + b21_ref[...]


def _edge_mlp(ea, w1, b1, w2, b2, scale):
    rows, din = ea.shape
    dh = w1.shape[1]
    zz = jnp.zeros_like(w1)
    w1p = jnp.block([[w1, zz], [zz, w1]])
    b1p = jnp.concatenate([b1, b1]).reshape(1, -1)
    z2 = jnp.zeros((dh, DH), jnp.float32)
    w20 = jnp.block([[w2[:, :DH] * scale[0], z2], [z2, w2[:, :DH] * scale[0]]])
    w21 = jnp.block([[w2[:, DH:] * scale[0], z2], [z2, w2[:, DH:] * scale[0]]])
    b20 = jnp.concatenate([b2[:DH], b2[:DH]]).reshape(1, -1) * scale[0]
    b21 = jnp.concatenate([b2[DH:], b2[DH:]]).reshape(1, -1) * scale[0]
    grid = rows // EBLK
    return pl.pallas_call(
        _edge_mlp_kernel,
        grid=(grid,),
        in_specs=[
            pl.BlockSpec((EBLK, din), lambda i: (i, 0)),
            pl.BlockSpec((din, 2 * dh), lambda i: (0, 0)),
            pl.BlockSpec((1, 2 * dh), lambda i: (0, 0)),
            pl.BlockSpec((2 * dh, D), lambda i: (0, 0)),
            pl.BlockSpec((2 * dh, D), lambda i: (0, 0)),
            pl.BlockSpec((1, D), lambda i: (0, 0)),
            pl.BlockSpec((1, D), lambda i: (0, 0)),
        ],
        out_specs=[pl.BlockSpec((EBLK, D), lambda i: (i, 0))] * 2,
        out_shape=[jax.ShapeDtypeStruct((rows, D), jnp.float32)] * 2,
    )(ea, w1p, b1p, w20, w21, b20, b21)


def _ln_kernel(x_ref, a0_ref, a1_ref, rw_ref, g_ref, b_ref, o_ref):
    agg = jnp.concatenate([a0_ref[...], a1_ref[...]], axis=1)
    out = x_ref[...] + agg * rw_ref[0, 0]
    mean = jnp.mean(out, axis=1, keepdims=True)
    cen = out - mean
    var = jnp.mean(cen * cen, axis=1, keepdims=True)
    o_ref[...] = cen * jax.lax.rsqrt(var + 1e-5) * g_ref[...] + b_ref[...]


def _ln_residual(x, a0, a1, rw, gamma, beta):
    rows = x.shape[0]
    return pl.pallas_call(
        _ln_kernel,
        grid=(rows // BLK,),
        in_specs=[
            pl.BlockSpec((BLK, D), lambda i: (i, 0)),
            pl.BlockSpec((BLK, DH), lambda i: (i, 0)),
            pl.BlockSpec((BLK, DH), lambda i: (i, 0)),
            pl.BlockSpec((1, 1), lambda i: (0, 0)),
            pl.BlockSpec((1, D), lambda i: (0, 0)),
            pl.BlockSpec((1, D), lambda i: (0, 0)),
        ],
        out_specs=pl.BlockSpec((BLK, D), lambda i: (i, 0)),
        out_shape=jax.ShapeDtypeStruct((rows, D), jnp.float32),
    )(x, a0, a1, rw.reshape(1, 1), gamma.reshape(1, -1), beta.reshape(1, -1))


HCH = CH // 2


def _sc_agg(xt0, xt1, te0, te1, sd):
    nchunk = sd.shape[0]
    pairs = ((nchunk + NS - 1) // NS + 1) // 2
    mesh = plsc.VectorSubcoreMesh(core_axis_name="c", subcore_axis_name="s")

    @functools.partial(
        pl.kernel,
        out_type=[jax.ShapeDtypeStruct((N_PAD, DH), jnp.float32)] * 2,
        mesh=mesh,
        scratch_types=[
            pltpu.VMEM((2, 2, CH), jnp.int32),
            pltpu.VMEM((2, CH, DH), jnp.float32),
            pltpu.VMEM((2, HCH, D), jnp.float32),
            pltpu.VMEM((CH, DH), jnp.float32),
            pltpu.VMEM_SHARED((N_PAD, DH), jnp.float32),
            pltpu.SemaphoreType.DMA,
            pltpu.SemaphoreType.DMA,
            pltpu.SemaphoreType.DMA,
            pltpu.SemaphoreType.DMA,
            pltpu.SemaphoreType.DMA,
            pltpu.SemaphoreType.DMA,
        ],
        compiler_params=pltpu.CompilerParams(use_tc_tiling_on_sc=False),
    )
    def k(xt0_hbm, xt1_hbm, te0_hbm, te1_hbm, sd_hbm,
          out0_hbm, out1_hbm, sdb, rows, tev, zbuf, table,
          si0, si1, sg0, sg1, st0, st1):
        c = lax.axis_index("c")
        s = lax.axis_index("s")
        sem_i = [si0, si1]
        sem_g = [sg0, sg1]
        sem_t = [st0, st1]

        @pl.loop(0, CH)
        def _(r):
            @pl.loop(0, DH // 16)
            def _(j):
                zbuf[r, pl.ds(j * 16, 16)] = jnp.zeros((16,), jnp.float32)

        @pl.loop(0, ROWS_PER_TILE // CH)
        def _(kk):
            pltpu.sync_copy(zbuf, table.at[pl.ds(s * ROWS_PER_TILE + kk * CH, CH)])

        plsc.subcore_barrier()

        def run(xt_hbm, te_hbm):
            def load_idx(b, g):
                pltpu.async_copy(sd_hbm.at[g], sdb.at[b], sem_i[b])

            def wait_idx(b):
                pltpu.make_async_copy(sd_hbm.at[0], sdb.at[b], sem_i[b]).wait()

            def load_data(b, g):
                pltpu.async_copy(xt_hbm.at[sdb.at[b, 0]], rows.at[b], sem_g[b])
                pltpu.async_copy(te_hbm.at[pl.ds(g * HCH, HCH)], tev.at[b],
                                 sem_t[b])

            def wait_data(b):
                pltpu.make_async_copy(xt_hbm.at[pl.ds(0, CH)], rows.at[b],
                                      sem_g[b]).wait()
                pltpu.make_async_copy(te_hbm.at[pl.ds(0, HCH)], tev.at[b],
                                      sem_t[b]).wait()

            def compute(b):
                @pl.loop(0, HCH)
                def _(r):
                    for p in range(2):
                        for j in range(DH // 16):
                            rows[b, 2 * r + p, pl.ds(j * 16, 16)] = (
                                rows[b, 2 * r + p, pl.ds(j * 16, 16)]
                                * tev[b, r, pl.ds(p * DH + j * 16, 16)])

                pltpu.sync_copy(rows.at[b], table.at[sdb.at[b, 1]], add=True)

            load_idx(0, s)

            @pl.when(NS + s < nchunk)
            def _():
                load_idx(1, NS + s)

            wait_idx(0)
            load_data(0, s)

            @pl.loop(0, pairs)
            def _(ip):
                for b in range(2):
                    g = (2 * ip + b) * NS + s
                    g1 = g + NS
                    g2 = g1 + NS

                    @pl.when(g < nchunk)
                    def _():
                        @pl.when(g1 < nchunk)
                        def _():
                            wait_idx(1 - b)
                            load_data(1 - b, g1)

                        wait_data(b)
                        compute(b)

                        @pl.when(g2 < nchunk)
                        def _():
                            load_idx(b, g2)

        @pl.when(c == 0)
        def _():
            run(xt0_hbm, te0_hbm)

        @pl.when(c == 1)
        def _():
            run(xt1_hbm, te1_hbm)

        plsc.subcore_barrier()

        def writeout(out_hbm):
            @pl.loop(0, ROWS_PER_TILE // CH)
            def _(kk):
                r0 = s * ROWS_PER_TILE + kk * CH
                pltpu.sync_copy(table.at[pl.ds(r0, CH)], out_hbm.at[pl.ds(r0, CH)])

        @pl.when(c == 0)
        def _():
            writeout(out0_hbm)

        @pl.when(c == 1)
        def _():
            writeout(out1_hbm)

    return k(xt0, xt1, te0, te1, sd)


def kernel(x, edge_index, edge_attr, nt_W1, nt_b1, nt_W2, nt_b2,
           et_W1, et_b1, et_W2, et_b2, residual_weight, dot_product_scale,
           ln_gamma, ln_beta):
    e = edge_index.shape[1]
    sd = edge_index.astype(jnp.int32).reshape(2, e // CH, CH).transpose(1, 0, 2)
    x_p = jnp.pad(x, ((0, N_PAD - N), (0, 0)))

    one = jnp.ones((1,), jnp.float32)
    xt0, xt1 = _mlp(x_p, nt_W1, nt_b1, nt_W2, nt_b2, one)
    ea_pair = edge_attr.reshape(e // 2, 2 * edge_attr.shape[1])
    te0, te1 = _edge_mlp(ea_pair, et_W1, et_b1, et_W2, et_b2,
                         dot_product_scale)

    agg0, agg1 = _sc_agg(xt0, xt1, te0, te1, sd)

    out = _ln_residual(x_p, agg0, agg1, residual_weight, ln_gamma, ln_beta)
    return out[:N]


# speedup vs baseline: 1.3680x; 1.0609x over previous
import functools

import jax
import jax.numpy as jnp
from jax import lax
from jax.experimental import pallas as pl
from jax.experimental.pallas import tpu as pltpu
from jax.experimental.pallas import tpu_sc as plsc

N = 10000
D = 128
N_PAD = 10240
BLK = 1024

NC = 2
NS = 16
CH = 128
DH = D // NC
ROWS_PER_TILE = N_PAD // NS


def _leaky(x):
    return jnp.where(x > 0, x, 0.1 * x)


def _dot3(a, b):
    return jnp.dot(a.astype(jnp.bfloat16), b.astype(jnp.bfloat16),
                   preferred_element_type=jnp.float32)


def _mlp_kernel(x_ref, w1_ref, b1_ref, w2_ref, b2_ref, s_ref, o0_ref, o1_ref):
    h = _leaky(_dot3(x_ref[...], w1_ref[...]) + b1_ref[...])
    o = _dot3(h, w2_ref[...]) + b2_ref[...]
    o = o * s_ref[0, 0]
    o0_ref[...] = o[:, :DH]
    o1_ref[...] = o[:, DH:]


def _mlp(x, w1, b1, w2, b2, scale):
    rows, din = x.shape
    dh = w1.shape[1]
    grid = rows // BLK
    return pl.pallas_call(
        _mlp_kernel,
        grid=(grid,),
        in_specs=[
            pl.BlockSpec((BLK, din), lambda i: (i, 0)),
            pl.BlockSpec((din, dh), lambda i: (0, 0)),
            pl.BlockSpec((1, dh), lambda i: (0, 0)),
            pl.BlockSpec((dh, D), lambda i: (0, 0)),
            pl.BlockSpec((1, D), lambda i: (0, 0)),
            pl.BlockSpec((1, 1), lambda i: (0, 0)),
        ],
        out_specs=[pl.BlockSpec((BLK, DH), lambda i: (i, 0))] * 2,
        out_shape=[jax.ShapeDtypeStruct((rows, DH), jnp.float32)] * 2,
    )(x, w1, b1.reshape(1, -1), w2, b2.reshape(1, -1), scale.reshape(1, 1))


EBLK = 2000


def _edge_mlp_kernel(x_ref, w1_ref, b1_ref, w20_ref, w21_ref, b20_ref,
                     b21_ref, o0_ref, o1_ref):
    h = _leaky(_dot3(x_ref[...], w1_ref[...]) + b1_ref[...])
    o0_ref[...] = _dot3(h, w20_ref[...]) + b20_ref[...]
    o1_ref[...] = _dot3(h, w21_ref[...]) + b21_ref[...]


def _edge_mlp(ea, w1, b1, w2, b2, scale):
    rows, din = ea.shape
    dh = w1.shape[1]
    zz = jnp.zeros_like(w1)
    w1p = jnp.block([[w1, zz], [zz, w1]])
    b1p = jnp.concatenate([b1, b1]).reshape(1, -1)
    z2 = jnp.zeros((dh, DH), jnp.float32)
    w20 = jnp.block([[w2[:, :DH] * scale[0], z2], [z2, w2[:, :DH] * scale[0]]])
    w21 = jnp.block([[w2[:, DH:] * scale[0], z2], [z2, w2[:, DH:] * scale[0]]])
    b20 = jnp.concatenate([b2[:DH], b2[:DH]]).reshape(1, -1) * scale[0]
    b21 = jnp.concatenate([b2[DH:], b2[DH:]]).reshape(1, -1) * scale[0]
    grid = rows // EBLK
    return pl.pallas_call(
        _edge_mlp_kernel,
        grid=(grid,),
        in_specs=[
            pl.BlockSpec((EBLK, din), lambda i: (i, 0)),
            pl.BlockSpec((din, 2 * dh), lambda i: (0, 0)),
            pl.BlockSpec((1, 2 * dh), lambda i: (0, 0)),
            pl.BlockSpec((2 * dh, D), lambda i: (0, 0)),
            pl.BlockSpec((2 * dh, D), lambda i: (0, 0)),
            pl.BlockSpec((1, D), lambda i: (0, 0)),
            pl.BlockSpec((1, D), lambda i: (0, 0)),
        ],
        out_specs=[pl.BlockSpec((EBLK, D), lambda i: (i, 0))] * 2,
        out_shape=[jax.ShapeDtypeStruct((rows, D), jnp.float32)] * 2,
    )(ea, w1p, b1p, w20, w21, b20, b21)


def _ln_kernel(x_ref, a0_ref, a1_ref, rw_ref, g_ref, b_ref, o_ref):
    agg = jnp.concatenate([a0_ref[...], a1_ref[...]], axis=1)
    out = x_ref[...] + agg * rw_ref[0, 0]
    mean = jnp.mean(out, axis=1, keepdims=True)
    cen = out - mean
    var = jnp.mean(cen * cen, axis=1, keepdims=True)
    o_ref[...] = cen * jax.lax.rsqrt(var + 1e-5) * g_ref[...] + b_ref[...]


def _ln_residual(x, a0, a1, rw, gamma, beta):
    rows = x.shape[0]
    return pl.pallas_call(
        _ln_kernel,
        grid=(rows // BLK,),
        in_specs=[
            pl.BlockSpec((BLK, D), lambda i: (i, 0)),
            pl.BlockSpec((BLK, DH), lambda i: (i, 0)),
            pl.BlockSpec((BLK, DH), lambda i: (i, 0)),
            pl.BlockSpec((1, 1), lambda i: (0, 0)),
            pl.BlockSpec((1, D), lambda i: (0, 0)),
            pl.BlockSpec((1, D), lambda i: (0, 0)),
        ],
        out_specs=pl.BlockSpec((BLK, D), lambda i: (i, 0)),
        out_shape=jax.ShapeDtypeStruct((rows, D), jnp.float32),
    )(x, a0, a1, rw.reshape(1, 1), gamma.reshape(1, -1), beta.reshape(1, -1))


HCH = CH // 2


def _sc_agg(xt0, xt1, te0, te1, sd):
    nchunk = sd.shape[0]
    pairs = ((nchunk + NS - 1) // NS + 1) // 2
    mesh = plsc.VectorSubcoreMesh(core_axis_name="c", subcore_axis_name="s")

    @functools.partial(
        pl.kernel,
        out_type=[jax.ShapeDtypeStruct((N_PAD, DH), jnp.float32)] * 2,
        mesh=mesh,
        scratch_types=[
            pltpu.VMEM((2, 2, CH), jnp.int32),
            pltpu.VMEM((2, CH, DH), jnp.float32),
            pltpu.VMEM((2, HCH, D), jnp.float32),
            pltpu.VMEM((CH, DH), jnp.float32),
            pltpu.VMEM_SHARED((N_PAD, DH), jnp.float32),
            pltpu.SemaphoreType.DMA,
            pltpu.SemaphoreType.DMA,
            pltpu.SemaphoreType.DMA,
            pltpu.SemaphoreType.DMA,
            pltpu.SemaphoreType.DMA,
            pltpu.SemaphoreType.DMA,
        ],
        compiler_params=pltpu.CompilerParams(use_tc_tiling_on_sc=False),
    )
    def k(xt0_hbm, xt1_hbm, te0_hbm, te1_hbm, sd_hbm,
          out0_hbm, out1_hbm, sdb, rows, tev, zbuf, table,
          si0, si1, sg0, sg1, st0, st1):
        c = lax.axis_index("c")
        s = lax.axis_index("s")
        sem_i = [si0, si1]
        sem_g = [sg0, sg1]
        sem_t = [st0, st1]

        @pl.loop(0, CH)
        def _(r):
            @pl.loop(0, DH // 16)
            def _(j):
                zbuf[r, pl.ds(j * 16, 16)] = jnp.zeros((16,), jnp.float32)

        @pl.loop(0, ROWS_PER_TILE // CH)
        def _(kk):
            pltpu.sync_copy(zbuf, table.at[pl.ds(s * ROWS_PER_TILE + kk * CH, CH)])

        plsc.subcore_barrier()

        def run(xt_hbm, te_hbm):
            def load_idx(b, g):
                pltpu.async_copy(sd_hbm.at[g], sdb.at[b], sem_i[b])

            def wait_idx(b):
                pltpu.make_async_copy(sd_hbm.at[0], sdb.at[b], sem_i[b]).wait()

            def load_data(b, g):
                pltpu.async_copy(xt_hbm.at[sdb.at[b, 0]], rows.at[b], sem_g[b])
                pltpu.async_copy(te_hbm.at[pl.ds(g * HCH, HCH)], tev.at[b],
                                 sem_t[b])

            def wait_data(b):
                pltpu.make_async_copy(xt_hbm.at[pl.ds(0, CH)], rows.at[b],
                                      sem_g[b]).wait()
                pltpu.make_async_copy(te_hbm.at[pl.ds(0, HCH)], tev.at[b],
                                      sem_t[b]).wait()

            def compute(b):
                @pl.loop(0, HCH)
                def _(r):
                    for p in range(2):
                        for j in range(DH // 16):
                            rows[b, 2 * r + p, pl.ds(j * 16, 16)] = (
                                rows[b, 2 * r + p, pl.ds(j * 16, 16)]
                                * tev[b, r, pl.ds(p * DH + j * 16, 16)])

                pltpu.sync_copy(rows.at[b], table.at[sdb.at[b, 1]], add=True)

            load_idx(0, s)

            @pl.when(NS + s < nchunk)
            def _():
                load_idx(1, NS + s)

            wait_idx(0)
            load_data(0, s)

            @pl.loop(0, pairs)
            def _(ip):
                for b in range(2):
                    g = (2 * ip + b) * NS + s
                    g1 = g + NS
                    g2 = g1 + NS

                    @pl.when(g < nchunk)
                    def _():
                        @pl.when(g1 < nchunk)
                        def _():
                            wait_idx(1 - b)
                            load_data(1 - b, g1)

                        wait_data(b)
                        compute(b)

                        @pl.when(g2 < nchunk)
                        def _():
                            load_idx(b, g2)

        @pl.when(c == 0)
        def _():
            run(xt0_hbm, te0_hbm)

        @pl.when(c == 1)
        def _():
            run(xt1_hbm, te1_hbm)

        plsc.subcore_barrier()

        def writeout(out_hbm):
            @pl.loop(0, ROWS_PER_TILE // CH)
            def _(kk):
                r0 = s * ROWS_PER_TILE + kk * CH
                pltpu.sync_copy(table.at[pl.ds(r0, CH)], out_hbm.at[pl.ds(r0, CH)])

        @pl.when(c == 0)
        def _():
            writeout(out0_hbm)

        @pl.when(c == 1)
        def _():
            writeout(out1_hbm)

    return k(xt0, xt1, te0, te1, sd)


def kernel(x, edge_index, edge_attr, nt_W1, nt_b1, nt_W2, nt_b2,
           et_W1, et_b1, et_W2, et_b2, residual_weight, dot_product_scale,
           ln_gamma, ln_beta):
    e = edge_index.shape[1]
    sd = edge_index.astype(jnp.int32).reshape(2, e // CH, CH).transpose(1, 0, 2)
    x_p = jnp.pad(x, ((0, N_PAD - N), (0, 0)))

    one = jnp.ones((1,), jnp.float32)
    xt0, xt1 = _mlp(x_p, nt_W1, nt_b1, nt_W2, nt_b2, one)
    ea_pair = edge_attr.reshape(e // 2, 2 * edge_attr.shape[1])
    te0, te1 = _edge_mlp(ea_pair, et_W1, et_b1, et_W2, et_b2,
                         dot_product_scale)

    agg0, agg1 = _sc_agg(xt0, xt1, te0, te1, sd)

    out = _ln_residual(x_p, agg0, agg1, residual_weight, ln_gamma, ln_beta)
    return out[:N]


# speedup vs baseline: 1.4269x; 1.0431x over previous
import functools

import jax
import jax.numpy as jnp
from jax import lax
from jax.experimental import pallas as pl
from jax.experimental.pallas import tpu as pltpu
from jax.experimental.pallas import tpu_sc as plsc

N = 10000
D = 128
N_PAD = 10240
BLK = 1024

NC = 2
NS = 16
CH = 128
DH = D // NC
ROWS_PER_TILE = N_PAD // NS


def _leaky(x):
    return jnp.where(x > 0, x, 0.1 * x)


def _dot3(a, b):
    return jnp.dot(a.astype(jnp.bfloat16), b.astype(jnp.bfloat16),
                   preferred_element_type=jnp.float32)


def _mlp_kernel(x_ref, w1_ref, b1_ref, w2_ref, b2_ref, s_ref, o0_ref, o1_ref):
    h = _leaky(_dot3(x_ref[...], w1_ref[...]) + b1_ref[...])
    o = _dot3(h, w2_ref[...]) + b2_ref[...]
    o = o * s_ref[0, 0]
    o0_ref[...] = o[:, :DH]
    o1_ref[...] = o[:, DH:]


def _mlp(x, w1, b1, w2, b2, scale):
    rows, din = x.shape
    dh = w1.shape[1]
    grid = rows // BLK
    return pl.pallas_call(
        _mlp_kernel,
        grid=(grid,),
        in_specs=[
            pl.BlockSpec((BLK, din), lambda i: (i, 0)),
            pl.BlockSpec((din, dh), lambda i: (0, 0)),
            pl.BlockSpec((1, dh), lambda i: (0, 0)),
            pl.BlockSpec((dh, D), lambda i: (0, 0)),
            pl.BlockSpec((1, D), lambda i: (0, 0)),
            pl.BlockSpec((1, 1), lambda i: (0, 0)),
        ],
        out_specs=[pl.BlockSpec((BLK, DH), lambda i: (i, 0))] * 2,
        out_shape=[jax.ShapeDtypeStruct((rows, DH), jnp.float32)] * 2,
    )(x, w1, b1.reshape(1, -1), w2, b2.reshape(1, -1), scale.reshape(1, 1))


EBLK = 4000


def _edge_mlp_kernel(x_ref, w1_ref, b1_ref, w20_ref, w21_ref, b20_ref,
                     b21_ref, o0_ref, o1_ref):
    h = _leaky(_dot3(x_ref[...], w1_ref[...]) + b1_ref[...])
    o0_ref[...] = _dot3(h, w20_ref[...]) + b20_ref[...]
    o1_ref[...] = _dot3(h, w21_ref[...]) + b21_ref[...]


def _edge_mlp(ea, w1, b1, w2, b2, scale):
    rows, din = ea.shape
    dh = w1.shape[1]
    zz = jnp.zeros_like(w1)
    w1p = jnp.block([[w1, zz], [zz, w1]])
    b1p = jnp.concatenate([b1, b1]).reshape(1, -1)
    z2 = jnp.zeros((dh, DH), jnp.float32)
    w20 = jnp.block([[w2[:, :DH] * scale[0], z2], [z2, w2[:, :DH] * scale[0]]])
    w21 = jnp.block([[w2[:, DH:] * scale[0], z2], [z2, w2[:, DH:] * scale[0]]])
    b20 = jnp.concatenate([b2[:DH], b2[:DH]]).reshape(1, -1) * scale[0]
    b21 = jnp.concatenate([b2[DH:], b2[DH:]]).reshape(1, -1) * scale[0]
    grid = rows // EBLK
    return pl.pallas_call(
        _edge_mlp_kernel,
        grid=(grid,),
        in_specs=[
            pl.BlockSpec((EBLK, din), lambda i: (i, 0)),
            pl.BlockSpec((din, 2 * dh), lambda i: (0, 0)),
            pl.BlockSpec((1, 2 * dh), lambda i: (0, 0)),
            pl.BlockSpec((2 * dh, D), lambda i: (0, 0)),
            pl.BlockSpec((2 * dh, D), lambda i: (0, 0)),
            pl.BlockSpec((1, D), lambda i: (0, 0)),
            pl.BlockSpec((1, D), lambda i: (0, 0)),
        ],
        out_specs=[pl.BlockSpec((EBLK, D), lambda i: (i, 0))] * 2,
        out_shape=[jax.ShapeDtypeStruct((rows, D), jnp.float32)] * 2,
    )(ea, w1p, b1p, w20, w21, b20, b21)


def _ln_kernel(x_ref, a0_ref, a1_ref, rw_ref, g_ref, b_ref, o_ref):
    agg = jnp.concatenate([a0_ref[...], a1_ref[...]], axis=1)
    out = x_ref[...] + agg * rw_ref[0, 0]
    mean = jnp.mean(out, axis=1, keepdims=True)
    cen = out - mean
    var = jnp.mean(cen * cen, axis=1, keepdims=True)
    o_ref[...] = cen * jax.lax.rsqrt(var + 1e-5) * g_ref[...] + b_ref[...]


def _ln_residual(x, a0, a1, rw, gamma, beta):
    rows = x.shape[0]
    return pl.pallas_call(
        _ln_kernel,
        grid=(rows // BLK,),
        in_specs=[
            pl.BlockSpec((BLK, D), lambda i: (i, 0)),
            pl.BlockSpec((BLK, DH), lambda i: (i, 0)),
            pl.BlockSpec((BLK, DH), lambda i: (i, 0)),
            pl.BlockSpec((1, 1), lambda i: (0, 0)),
            pl.BlockSpec((1, D), lambda i: (0, 0)),
            pl.BlockSpec((1, D), lambda i: (0, 0)),
        ],
        out_specs=pl.BlockSpec((BLK, D), lambda i: (i, 0)),
        out_shape=jax.ShapeDtypeStruct((rows, D), jnp.float32),
    )(x, a0, a1, rw.reshape(1, 1), gamma.reshape(1, -1), beta.reshape(1, -1))


HCH = CH // 2


def _sc_agg(xt0, xt1, te0, te1, sd):
    nchunk = sd.shape[0]
    pairs = ((nchunk + NS - 1) // NS + 1) // 2
    mesh = plsc.VectorSubcoreMesh(core_axis_name="c", subcore_axis_name="s")

    @functools.partial(
        pl.kernel,
        out_type=[jax.ShapeDtypeStruct((N_PAD, DH), jnp.float32)] * 2,
        mesh=mesh,
        scratch_types=[
            pltpu.VMEM((2, 2, CH), jnp.int32),
            pltpu.VMEM((2, CH, DH), jnp.float32),
            pltpu.VMEM((2, HCH, D), jnp.float32),
            pltpu.VMEM((CH, DH), jnp.float32),
            pltpu.VMEM_SHARED((N_PAD, DH), jnp.float32),
            pltpu.SemaphoreType.DMA,
            pltpu.SemaphoreType.DMA,
            pltpu.SemaphoreType.DMA,
            pltpu.SemaphoreType.DMA,
            pltpu.SemaphoreType.DMA,
            pltpu.SemaphoreType.DMA,
        ],
        compiler_params=pltpu.CompilerParams(use_tc_tiling_on_sc=False),
    )
    def k(xt0_hbm, xt1_hbm, te0_hbm, te1_hbm, sd_hbm,
          out0_hbm, out1_hbm, sdb, rows, tev, zbuf, table,
          si0, si1, sg0, sg1, st0, st1):
        c = lax.axis_index("c")
        s = lax.axis_index("s")
        sem_i = [si0, si1]
        sem_g = [sg0, sg1]
        sem_t = [st0, st1]

        @pl.loop(0, CH)
        def _(r):
            @pl.loop(0, DH // 16)
            def _(j):
                zbuf[r, pl.ds(j * 16, 16)] = jnp.zeros((16,), jnp.float32)

        @pl.loop(0, ROWS_PER_TILE // CH)
        def _(kk):
            pltpu.sync_copy(zbuf, table.at[pl.ds(s * ROWS_PER_TILE + kk * CH, CH)])

        plsc.subcore_barrier()

        def run(xt_hbm, te_hbm):
            def load_idx(b, g):
                pltpu.async_copy(sd_hbm.at[g], sdb.at[b], sem_i[b])

            def wait_idx(b):
                pltpu.make_async_copy(sd_hbm.at[0], sdb.at[b], sem_i[b]).wait()

            def load_data(b, g):
                pltpu.async_copy(xt_hbm.at[sdb.at[b, 0]], rows.at[b], sem_g[b])
                pltpu.async_copy(te_hbm.at[pl.ds(g * HCH, HCH)], tev.at[b],
                                 sem_t[b])

            def wait_data(b):
                pltpu.make_async_copy(xt_hbm.at[pl.ds(0, CH)], rows.at[b],
                                      sem_g[b]).wait()
                pltpu.make_async_copy(te_hbm.at[pl.ds(0, HCH)], tev.at[b],
                                      sem_t[b]).wait()

            def compute(b):
                @pl.loop(0, HCH)
                def _(r):
                    for p in range(2):
                        for j in range(DH // 16):
                            rows[b, 2 * r + p, pl.ds(j * 16, 16)] = (
                                rows[b, 2 * r + p, pl.ds(j * 16, 16)]
                                * tev[b, r, pl.ds(p * DH + j * 16, 16)])

                pltpu.sync_copy(rows.at[b], table.at[sdb.at[b, 1]], add=True)

            load_idx(0, s)

            @pl.when(NS + s < nchunk)
            def _():
                load_idx(1, NS + s)

            wait_idx(0)
            load_data(0, s)

            @pl.loop(0, pairs)
            def _(ip):
                for b in range(2):
                    g = (2 * ip + b) * NS + s
                    g1 = g + NS
                    g2 = g1 + NS

                    @pl.when(g < nchunk)
                    def _():
                        @pl.when(g1 < nchunk)
                        def _():
                            wait_idx(1 - b)
                            load_data(1 - b, g1)

                        wait_data(b)
                        compute(b)

                        @pl.when(g2 < nchunk)
                        def _():
                            load_idx(b, g2)

        @pl.when(c == 0)
        def _():
            run(xt0_hbm, te0_hbm)

        @pl.when(c == 1)
        def _():
            run(xt1_hbm, te1_hbm)

        plsc.subcore_barrier()

        def writeout(out_hbm):
            @pl.loop(0, ROWS_PER_TILE // CH)
            def _(kk):
                r0 = s * ROWS_PER_TILE + kk * CH
                pltpu.sync_copy(table.at[pl.ds(r0, CH)], out_hbm.at[pl.ds(r0, CH)])

        @pl.when(c == 0)
        def _():
            writeout(out0_hbm)

        @pl.when(c == 1)
        def _():
            writeout(out1_hbm)

    return k(xt0, xt1, te0, te1, sd)


def kernel(x, edge_index, edge_attr, nt_W1, nt_b1, nt_W2, nt_b2,
           et_W1, et_b1, et_W2, et_b2, residual_weight, dot_product_scale,
           ln_gamma, ln_beta):
    e = edge_index.shape[1]
    sd = edge_index.astype(jnp.int32).reshape(2, e // CH, CH).transpose(1, 0, 2)
    x_p = jnp.pad(x, ((0, N_PAD - N), (0, 0)))

    one = jnp.ones((1,), jnp.float32)
    xt0, xt1 = _mlp(x_p, nt_W1, nt_b1, nt_W2, nt_b2, one)
    ea_pair = edge_attr.reshape(e // 2, 2 * edge_attr.shape[1])
    te0, te1 = _edge_mlp(ea_pair, et_W1, et_b1, et_W2, et_b2,
                         dot_product_scale)

    agg0, agg1 = _sc_agg(xt0, xt1, te0, te1, sd)

    out = _ln_residual(x_p, agg0, agg1, residual_weight, ln_gamma, ln_beta)
    return out[:N]


# speedup vs baseline: 1.4592x; 1.0226x over previous
import functools

import jax
import jax.numpy as jnp
from jax import lax
from jax.experimental import pallas as pl
from jax.experimental.pallas import tpu as pltpu
from jax.experimental.pallas import tpu_sc as plsc

N = 10000
D = 128
N_PAD = 10240
BLK = 1024

NC = 2
NS = 16
CH = 128
DH = D // NC
ROWS_PER_TILE = N_PAD // NS


def _leaky(x):
    return jnp.where(x > 0, x, 0.1 * x)


def _dot3(a, b):
    return jnp.dot(a.astype(jnp.bfloat16), b.astype(jnp.bfloat16),
                   preferred_element_type=jnp.float32)


def _mlp_kernel(x_ref, w1_ref, b1_ref, w2_ref, b2_ref, s_ref, o0_ref, o1_ref):
    h = _leaky(_dot3(x_ref[...], w1_ref[...]) + b1_ref[...])
    o = _dot3(h, w2_ref[...]) + b2_ref[...]
    o = o * s_ref[0, 0]
    o0_ref[...] = o[:, :DH]
    o1_ref[...] = o[:, DH:]


def _mlp(x, w1, b1, w2, b2, scale):
    rows, din = x.shape
    dh = w1.shape[1]
    grid = rows // BLK
    return pl.pallas_call(
        _mlp_kernel,
        grid=(grid,),
        in_specs=[
            pl.BlockSpec((BLK, din), lambda i: (i, 0)),
            pl.BlockSpec((din, dh), lambda i: (0, 0)),
            pl.BlockSpec((1, dh), lambda i: (0, 0)),
            pl.BlockSpec((dh, D), lambda i: (0, 0)),
            pl.BlockSpec((1, D), lambda i: (0, 0)),
            pl.BlockSpec((1, 1), lambda i: (0, 0)),
        ],
        out_specs=[pl.BlockSpec((BLK, DH), lambda i: (i, 0))] * 2,
        out_shape=[jax.ShapeDtypeStruct((rows, DH), jnp.float32)] * 2,
    )(x, w1, b1.reshape(1, -1), w2, b2.reshape(1, -1), scale.reshape(1, 1))


EBLK = 8000


def _edge_mlp_kernel(x_ref, w1_ref, b1_ref, w20_ref, w21_ref, b20_ref,
                     b21_ref, o0_ref, o1_ref):
    h = _leaky(_dot3(x_ref[...], w1_ref[...]) + b1_ref[...])
    o0_ref[...] = _dot3(h, w20_ref[...]) + b20_ref[...]
    o1_ref[...] = _dot3(h, w21_ref[...]) + b21_ref[...]


def _edge_mlp(ea, w1, b1, w2, b2, scale):
    rows, din = ea.shape
    dh = w1.shape[1]
    zz = jnp.zeros_like(w1)
    w1p = jnp.block([[w1, zz], [zz, w1]])
    b1p = jnp.concatenate([b1, b1]).reshape(1, -1)
    z2 = jnp.zeros((dh, DH), jnp.float32)
    w20 = jnp.block([[w2[:, :DH] * scale[0], z2], [z2, w2[:, :DH] * scale[0]]])
    w21 = jnp.block([[w2[:, DH:] * scale[0], z2], [z2, w2[:, DH:] * scale[0]]])
    b20 = jnp.concatenate([b2[:DH], b2[:DH]]).reshape(1, -1) * scale[0]
    b21 = jnp.concatenate([b2[DH:], b2[DH:]]).reshape(1, -1) * scale[0]
    grid = rows // EBLK
    return pl.pallas_call(
        _edge_mlp_kernel,
        grid=(grid,),
        in_specs=[
            pl.BlockSpec((EBLK, din), lambda i: (i, 0)),
            pl.BlockSpec((din, 2 * dh), lambda i: (0, 0)),
            pl.BlockSpec((1, 2 * dh), lambda i: (0, 0)),
            pl.BlockSpec((2 * dh, D), lambda i: (0, 0)),
            pl.BlockSpec((2 * dh, D), lambda i: (0, 0)),
            pl.BlockSpec((1, D), lambda i: (0, 0)),
            pl.BlockSpec((1, D), lambda i: (0, 0)),
        ],
        out_specs=[pl.BlockSpec((EBLK, D), lambda i: (i, 0))] * 2,
        out_shape=[jax.ShapeDtypeStruct((rows, D), jnp.float32)] * 2,
    )(ea, w1p, b1p, w20, w21, b20, b21)


def _ln_kernel(x_ref, a0_ref, a1_ref, rw_ref, g_ref, b_ref, o_ref):
    agg = jnp.concatenate([a0_ref[...], a1_ref[...]], axis=1)
    out = x_ref[...] + agg * rw_ref[0, 0]
    mean = jnp.mean(out, axis=1, keepdims=True)
    cen = out - mean
    var = jnp.mean(cen * cen, axis=1, keepdims=True)
    o_ref[...] = cen * jax.lax.rsqrt(var + 1e-5) * g_ref[...] + b_ref[...]


def _ln_residual(x, a0, a1, rw, gamma, beta):
    rows = x.shape[0]
    return pl.pallas_call(
        _ln_kernel,
        grid=(rows // BLK,),
        in_specs=[
            pl.BlockSpec((BLK, D), lambda i: (i, 0)),
            pl.BlockSpec((BLK, DH), lambda i: (i, 0)),
            pl.BlockSpec((BLK, DH), lambda i: (i, 0)),
            pl.BlockSpec((1, 1), lambda i: (0, 0)),
            pl.BlockSpec((1, D), lambda i: (0, 0)),
            pl.BlockSpec((1, D), lambda i: (0, 0)),
        ],
        out_specs=pl.BlockSpec((BLK, D), lambda i: (i, 0)),
        out_shape=jax.ShapeDtypeStruct((rows, D), jnp.float32),
    )(x, a0, a1, rw.reshape(1, 1), gamma.reshape(1, -1), beta.reshape(1, -1))


HCH = CH // 2


def _sc_agg(xt0, xt1, te0, te1, sd):
    nchunk = sd.shape[0]
    pairs = ((nchunk + NS - 1) // NS + 1) // 2
    mesh = plsc.VectorSubcoreMesh(core_axis_name="c", subcore_axis_name="s")

    @functools.partial(
        pl.kernel,
        out_type=[jax.ShapeDtypeStruct((N_PAD, DH), jnp.float32)] * 2,
        mesh=mesh,
        scratch_types=[
            pltpu.VMEM((2, 2, CH), jnp.int32),
            pltpu.VMEM((2, CH, DH), jnp.float32),
            pltpu.VMEM((2, HCH, D), jnp.float32),
            pltpu.VMEM((CH, DH), jnp.float32),
            pltpu.VMEM_SHARED((N_PAD, DH), jnp.float32),
            pltpu.SemaphoreType.DMA,
            pltpu.SemaphoreType.DMA,
            pltpu.SemaphoreType.DMA,
            pltpu.SemaphoreType.DMA,
            pltpu.SemaphoreType.DMA,
            pltpu.SemaphoreType.DMA,
        ],
        compiler_params=pltpu.CompilerParams(use_tc_tiling_on_sc=False),
    )
    def k(xt0_hbm, xt1_hbm, te0_hbm, te1_hbm, sd_hbm,
          out0_hbm, out1_hbm, sdb, rows, tev, zbuf, table,
          si0, si1, sg0, sg1, st0, st1):
        c = lax.axis_index("c")
        s = lax.axis_index("s")
        sem_i = [si0, si1]
        sem_g = [sg0, sg1]
        sem_t = [st0, st1]

        @pl.loop(0, CH)
        def _(r):
            @pl.loop(0, DH // 16)
            def _(j):
                zbuf[r, pl.ds(j * 16, 16)] = jnp.zeros((16,), jnp.float32)

        @pl.loop(0, ROWS_PER_TILE // CH)
        def _(kk):
            pltpu.sync_copy(zbuf, table.at[pl.ds(s * ROWS_PER_TILE + kk * CH, CH)])

        plsc.subcore_barrier()

        def run(xt_hbm, te_hbm):
            def load_idx(b, g):
                pltpu.async_copy(sd_hbm.at[g], sdb.at[b], sem_i[b])

            def wait_idx(b):
                pltpu.make_async_copy(sd_hbm.at[0], sdb.at[b], sem_i[b]).wait()

            def load_data(b, g):
                pltpu.async_copy(xt_hbm.at[sdb.at[b, 0]], rows.at[b], sem_g[b])
                pltpu.async_copy(te_hbm.at[pl.ds(g * HCH, HCH)], tev.at[b],
                                 sem_t[b])

            def wait_data(b):
                pltpu.make_async_copy(xt_hbm.at[pl.ds(0, CH)], rows.at[b],
                                      sem_g[b]).wait()
                pltpu.make_async_copy(te_hbm.at[pl.ds(0, HCH)], tev.at[b],
                                      sem_t[b]).wait()

            def compute(b):
                @pl.loop(0, HCH)
                def _(r):
                    for p in range(2):
                        for j in range(DH // 16):
                            rows[b, 2 * r + p, pl.ds(j * 16, 16)] = (
                                rows[b, 2 * r + p, pl.ds(j * 16, 16)]
                                * tev[b, r, pl.ds(p * DH + j * 16, 16)])

                pltpu.sync_copy(rows.at[b], table.at[sdb.at[b, 1]], add=True)

            load_idx(0, s)

            @pl.when(NS + s < nchunk)
            def _():
                load_idx(1, NS + s)

            wait_idx(0)
            load_data(0, s)

            @pl.loop(0, pairs)
            def _(ip):
                for b in range(2):
                    g = (2 * ip + b) * NS + s
                    g1 = g + NS
                    g2 = g1 + NS

                    @pl.when(g < nchunk)
                    def _():
                        @pl.when(g1 < nchunk)
                        def _():
                            wait_idx(1 - b)
                            load_data(1 - b, g1)

                        wait_data(b)
                        compute(b)

                        @pl.when(g2 < nchunk)
                        def _():
                            load_idx(b, g2)

        @pl.when(c == 0)
        def _():
            run(xt0_hbm, te0_hbm)

        @pl.when(c == 1)
        def _():
            run(xt1_hbm, te1_hbm)

        plsc.subcore_barrier()

        def writeout(out_hbm):
            @pl.loop(0, ROWS_PER_TILE // CH)
            def _(kk):
                r0 = s * ROWS_PER_TILE + kk * CH
                pltpu.sync_copy(table.at[pl.ds(r0, CH)], out_hbm.at[pl.ds(r0, CH)])

        @pl.when(c == 0)
        def _():
            writeout(out0_hbm)

        @pl.when(c == 1)
        def _():
            writeout(out1_hbm)

    return k(xt0, xt1, te0, te1, sd)


def kernel(x, edge_index, edge_attr, nt_W1, nt_b1, nt_W2, nt_b2,
           et_W1, et_b1, et_W2, et_b2, residual_weight, dot_product_scale,
           ln_gamma, ln_beta):
    e = edge_index.shape[1]
    sd = edge_index.astype(jnp.int32).reshape(2, e // CH, CH).transpose(1, 0, 2)
    x_p = jnp.pad(x, ((0, N_PAD - N), (0, 0)))

    one = jnp.ones((1,), jnp.float32)
    xt0, xt1 = _mlp(x_p, nt_W1, nt_b1, nt_W2, nt_b2, one)
    ea_pair = edge_attr.reshape(e // 2, 2 * edge_attr.shape[1])
    te0, te1 = _edge_mlp(ea_pair, et_W1, et_b1, et_W2, et_b2,
                         dot_product_scale)

    agg0, agg1 = _sc_agg(xt0, xt1, te0, te1, sd)

    out = _ln_residual(x_p, agg0, agg1, residual_weight, ln_gamma, ln_beta)
    return out[:N]
